# baseline (device time: 32959 ns/iter reference)
import jax
import jax.numpy as jnp
from jax import lax
from jax.experimental import pallas as pl
from jax.experimental.pallas import tpu as pltpu

N_DEV = 8
N_LAYERS = 3
MASKS = (1, 3, 4)


def kernel(x, Win0, Wout0, Win1, Wout1, Win2, Wout2):
    b, d_model = x.shape

    def body(x_ref, win0_ref, wout0_ref, win1_ref, wout1_ref,
             win2_ref, wout2_ref, out_ref, src_ref, dst_ref,
             send_sems, recv_sems):
        my_i = lax.axis_index("i")

        barrier_sem = pltpu.get_barrier_semaphore()
        for mask in MASKS:
            partner = jnp.bitwise_xor(my_i, mask)
            pl.semaphore_signal(
                barrier_sem, inc=1,
                device_id=(partner,), device_id_type=pl.DeviceIdType.MESH,
            )
        pl.semaphore_wait(barrier_sem, len(MASKS))

        wins = [win0_ref, win1_ref, win2_ref]
        wouts = [wout0_ref, wout1_ref, wout2_ref]

        x_cur = x_ref[:, :]
        for layer in range(N_LAYERS):
            h = jnp.maximum(
                jnp.dot(x_cur, wins[layer][:, :],
                        preferred_element_type=jnp.float32),
                0.0,
            )
            acc = jnp.dot(h, wouts[layer][:, :],
                          preferred_element_type=jnp.float32)

            for r, mask in enumerate(MASKS):
                partner = jnp.bitwise_xor(my_i, mask)
                src_ref[layer, r] = acc
                rdma = pltpu.make_async_remote_copy(
                    src_ref=src_ref.at[layer, r],
                    dst_ref=dst_ref.at[layer, r],
                    send_sem=send_sems.at[layer, r],
                    recv_sem=recv_sems.at[layer, r],
                    device_id=(partner,),
                    device_id_type=pl.DeviceIdType.MESH,
                )
                rdma.start()
                rdma.wait_recv()
                acc = acc + dst_ref[layer, r]
                rdma.wait_send()

            x_cur = acc

        out_ref[:, :] = x_cur

    return pl.pallas_call(
        body,
        out_shape=jax.ShapeDtypeStruct((b, d_model), jnp.float32),
        in_specs=[pl.BlockSpec(memory_space=pltpu.VMEM)] * 7,
        out_specs=pl.BlockSpec(memory_space=pltpu.VMEM),
        scratch_shapes=[
            pltpu.VMEM((N_LAYERS, len(MASKS), b, d_model), jnp.float32),
            pltpu.VMEM((N_LAYERS, len(MASKS), b, d_model), jnp.float32),
            pltpu.SemaphoreType.DMA((N_LAYERS, len(MASKS))),
            pltpu.SemaphoreType.DMA((N_LAYERS, len(MASKS))),
        ],
        compiler_params=pltpu.CompilerParams(collective_id=0),
    )(x, Win0, Wout0, Win1, Wout1, Win2, Wout2)


# device time: 26925 ns/iter; 1.2241x vs baseline; 1.2241x over previous
import jax
import jax.numpy as jnp
from jax import lax
from jax.experimental import pallas as pl
from jax.experimental.pallas import tpu as pltpu

N_DEV = 8
N_LAYERS = 3
SEND_ORDER = (6, 2, 5, 7, 1, 3, 4)
WAIT_ORDER = (1, 3, 4, 2, 5, 7, 6)


def kernel(x, Win0, Wout0, Win1, Wout1, Win2, Wout2):
    b, d_model = x.shape

    def body(x_ref, win0_ref, wout0_ref, win1_ref, wout1_ref,
             win2_ref, wout2_ref, out_ref, comm_ref, send_sems, recv_sems):
        my_i = lax.axis_index("i")

        barrier_sem = pltpu.get_barrier_semaphore()
        for m in range(1, N_DEV):
            peer = jnp.bitwise_xor(my_i, m)
            pl.semaphore_signal(
                barrier_sem, inc=1,
                device_id=(peer,), device_id_type=pl.DeviceIdType.MESH,
            )
        pl.semaphore_wait(barrier_sem, N_DEV - 1)

        wins = [win0_ref, win1_ref, win2_ref]
        wouts = [wout0_ref, wout1_ref, wout2_ref]

        x_cur = x_ref[:, :]
        for layer in range(N_LAYERS):
            h = jnp.maximum(
                jnp.dot(x_cur, wins[layer][:, :],
                        preferred_element_type=jnp.float32),
                0.0,
            )
            partial = jnp.dot(h, wouts[layer][:, :],
                              preferred_element_type=jnp.float32)
            comm_ref[layer, 0] = partial

            rdmas = {}
            for m in SEND_ORDER:
                peer = jnp.bitwise_xor(my_i, m)
                rdma = pltpu.make_async_remote_copy(
                    src_ref=comm_ref.at[layer, 0],
                    dst_ref=comm_ref.at[layer, m],
                    send_sem=send_sems.at[layer, m],
                    recv_sem=recv_sems.at[layer, m],
                    device_id=(peer,),
                    device_id_type=pl.DeviceIdType.MESH,
                )
                rdma.start()
                rdmas[m] = rdma

            acc = partial
            for m in WAIT_ORDER:
                rdmas[m].wait_recv()
                acc = acc + comm_ref[layer, m]
            for m in WAIT_ORDER:
                rdmas[m].wait_send()
            x_cur = acc

        out_ref[:, :] = x_cur

    return pl.pallas_call(
        body,
        out_shape=jax.ShapeDtypeStruct((b, d_model), jnp.float32),
        in_specs=[pl.BlockSpec(memory_space=pltpu.VMEM)] * 7,
        out_specs=pl.BlockSpec(memory_space=pltpu.VMEM),
        scratch_shapes=[
            pltpu.VMEM((N_LAYERS, N_DEV, b, d_model), jnp.float32),
            pltpu.SemaphoreType.DMA((N_LAYERS, N_DEV)),
            pltpu.SemaphoreType.DMA((N_LAYERS, N_DEV)),
        ],
        compiler_params=pltpu.CompilerParams(collective_id=0),
    )(x, Win0, Wout0, Win1, Wout1, Win2, Wout2)
